# baseline (speedup 1.0000x reference)
import functools

import jax
import jax.numpy as jnp
import numpy as np
from jax.experimental import pallas as pl
from jax.experimental.pallas import tpu as pltpu

_BASE_K = 64
_MIN_K = 16
_MAX_K = 512

_ROWS = 256
_CT = 256


def _mask_kernel(k_fixed, scores_ref, mask_ref, sk_ref):
    rows, s = scores_ref.shape[1], scores_ref.shape[2]
    ct = _CT
    j = pl.program_id(1)
    x = scores_ref[0]

    b = jax.lax.bitcast_convert_type(x, jnp.int32)
    sk = jnp.where(b < 0, b ^ jnp.int32(0x7FFFFFFF), b)

    q = j * rows + jax.lax.broadcasted_iota(jnp.int32, (rows, s), 0)
    c = jax.lax.broadcasted_iota(jnp.int32, (rows, s), 1)
    neg = jnp.int32(-(2**31))
    sk_ref[...] = jnp.where(c <= q, sk, neg)

    k_eff = jnp.minimum(jnp.int32(k_fixed), q[:, :1] + 1)

    sign = jnp.int32(-(2**31))
    ntiles = (j + 1) * (rows // ct)

    def body(i, t):
        cand = t | (jnp.int32(1) << (31 - i))
        thresh = cand ^ sign

        def tile_body(tc, acc):
            tile = sk_ref[:, pl.ds(tc * ct, ct)]
            return acc + jnp.sum((tile >= thresh).astype(jnp.int32),
                                 axis=1, keepdims=True)

        cnt = jax.lax.fori_loop(0, ntiles, tile_body,
                                jnp.zeros((rows, 1), jnp.int32))
        return jnp.where(cnt >= k_eff, cand, t)

    t = jax.lax.fori_loop(0, 32, body, jnp.zeros((rows, 1), jnp.int32))
    thresh = t ^ sign
    mask_ref[0] = sk_ref[...] >= thresh


@functools.partial(jax.jit, static_argnames=())
def kernel(x, index_scores, Wq, Wk):
    bsz, s, _ = index_scores.shape
    k_fixed = min(_BASE_K, s)
    k_fixed = int(np.clip(k_fixed, _MIN_K, min(_MAX_K, s)))

    rows = min(_ROWS, s)
    grid = (bsz, s // rows)
    mask = pl.pallas_call(
        functools.partial(_mask_kernel, k_fixed),
        grid=grid,
        in_specs=[pl.BlockSpec((1, rows, s), lambda b, r: (b, r, 0))],
        out_specs=pl.BlockSpec((1, rows, s), lambda b, r: (b, r, 0)),
        out_shape=jax.ShapeDtypeStruct((bsz, s, s), jnp.bool_),
        scratch_shapes=[pltpu.VMEM((rows, s), jnp.int32)],
    )(index_scores)

    k_values = jnp.full((bsz, s), k_fixed, dtype=jnp.int32)
    return (mask, k_values)


# speedup vs baseline: 2.2568x; 2.2568x over previous
import functools

import jax
import jax.numpy as jnp
import numpy as np
from jax.experimental import pallas as pl

_BASE_K = 64
_MIN_K = 16
_MAX_K = 512

_ROWS = 512


def _mask_kernel(k_fixed, scores_ref, mask_ref):
    rows, s = scores_ref.shape[1], scores_ref.shape[2]
    j = pl.program_id(1)
    x = scores_ref[0]

    b = jax.lax.bitcast_convert_type(x, jnp.int32)
    sk = jnp.where(b < 0, b ^ jnp.int32(0x7FFFFFFF), b)

    q = j * rows + jax.lax.broadcasted_iota(jnp.int32, (rows, s), 0)
    c = jax.lax.broadcasted_iota(jnp.int32, (rows, s), 1)
    neg = jnp.int32(-(2**31))
    sk = jnp.where(c <= q, sk, neg)

    k_eff = jnp.minimum(jnp.int32(k_fixed), q[:, :1] + 1)

    sign = jnp.int32(-(2**31))

    def body(i, t):
        cand = t | (jnp.int32(1) << (31 - i))
        thresh = cand ^ sign
        cnt = jnp.sum((sk >= thresh).astype(jnp.int32), axis=1, keepdims=True)
        return jnp.where(cnt >= k_eff, cand, t)

    t = jax.lax.fori_loop(0, 32, body, jnp.zeros((rows, 1), jnp.int32))
    thresh = t ^ sign
    mask_ref[0] = sk >= thresh


@functools.partial(jax.jit, static_argnames=())
def kernel(x, index_scores, Wq, Wk):
    bsz, s, _ = index_scores.shape
    k_fixed = min(_BASE_K, s)
    k_fixed = int(np.clip(k_fixed, _MIN_K, min(_MAX_K, s)))

    rows = min(_ROWS, s)
    grid = (bsz, s // rows)
    mask = pl.pallas_call(
        functools.partial(_mask_kernel, k_fixed),
        grid=grid,
        in_specs=[pl.BlockSpec((1, rows, s), lambda b, r: (b, r, 0))],
        out_specs=pl.BlockSpec((1, rows, s), lambda b, r: (b, r, 0)),
        out_shape=jax.ShapeDtypeStruct((bsz, s, s), jnp.bool_),
    )(index_scores)

    k_values = jnp.full((bsz, s), k_fixed, dtype=jnp.int32)
    return (mask, k_values)


# speedup vs baseline: 2.3224x; 1.0291x over previous
import functools

import jax
import jax.numpy as jnp
import numpy as np
from jax.experimental import pallas as pl

_BASE_K = 64
_MIN_K = 16
_MAX_K = 512

_ROWS = 1024


def _mask_kernel(k_fixed, scores_ref, mask_ref):
    rows, s = scores_ref.shape[1], scores_ref.shape[2]
    j = pl.program_id(1)
    x = scores_ref[0]

    b = jax.lax.bitcast_convert_type(x, jnp.int32)
    sk = jnp.where(b < 0, b ^ jnp.int32(0x7FFFFFFF), b)

    q = j * rows + jax.lax.broadcasted_iota(jnp.int32, (rows, s), 0)
    c = jax.lax.broadcasted_iota(jnp.int32, (rows, s), 1)
    neg = jnp.int32(-(2**31))
    sk = jnp.where(c <= q, sk, neg)

    k_eff = jnp.minimum(jnp.int32(k_fixed), q[:, :1] + 1)

    sign = jnp.int32(-(2**31))

    def body(i, t):
        cand = t | (jnp.int32(1) << (31 - i))
        thresh = cand ^ sign
        cnt = jnp.sum((sk >= thresh).astype(jnp.int32), axis=1, keepdims=True)
        return jnp.where(cnt >= k_eff, cand, t)

    t = jax.lax.fori_loop(0, 32, body, jnp.zeros((rows, 1), jnp.int32))
    thresh = t ^ sign
    mask_ref[0] = sk >= thresh


@functools.partial(jax.jit, static_argnames=())
def kernel(x, index_scores, Wq, Wk):
    bsz, s, _ = index_scores.shape
    k_fixed = min(_BASE_K, s)
    k_fixed = int(np.clip(k_fixed, _MIN_K, min(_MAX_K, s)))

    rows = min(_ROWS, s)
    grid = (bsz, s // rows)
    mask = pl.pallas_call(
        functools.partial(_mask_kernel, k_fixed),
        grid=grid,
        in_specs=[pl.BlockSpec((1, rows, s), lambda b, r: (b, r, 0))],
        out_specs=pl.BlockSpec((1, rows, s), lambda b, r: (b, r, 0)),
        out_shape=jax.ShapeDtypeStruct((bsz, s, s), jnp.bool_),
    )(index_scores)

    k_values = jnp.full((bsz, s), k_fixed, dtype=jnp.int32)
    return (mask, k_values)


# speedup vs baseline: 3.0309x; 1.3051x over previous
import functools

import jax
import jax.numpy as jnp
import numpy as np
from jax.experimental import pallas as pl

_BASE_K = 64
_MIN_K = 16
_MAX_K = 512

_CHUNK = 512


def _mask_kernel(k_fixed, nchunks, *refs):
    mask_ref = refs[-1]
    s = mask_ref.shape[2]
    j = pl.program_id(1)
    neg = jnp.int32(-(2**31))
    sign = jnp.int32(-(2**31))

    for i in range(nchunks):

        @pl.when(j == i)
        def _(i=i):
            sref = refs[i]
            rows, width = sref.shape[1], sref.shape[2]
            r0 = i * rows
            x = sref[0]

            b = jax.lax.bitcast_convert_type(x, jnp.int32)
            sk = jnp.where(b < 0, b ^ jnp.int32(0x7FFFFFFF), b)

            q = r0 + jax.lax.broadcasted_iota(jnp.int32, (rows, width), 0)
            c = jax.lax.broadcasted_iota(jnp.int32, (rows, width), 1)
            sk = jnp.where(c <= q, sk, neg)

            if r0 + 1 >= k_fixed:
                k_eff = jnp.full((rows, 1), k_fixed, jnp.int32)
            else:
                k_eff = jnp.minimum(jnp.int32(k_fixed), q[:, :1] + 1)

            def body(it, t):
                cand = t | (jnp.int32(1) << (31 - it))
                thresh = cand ^ sign
                cnt = jnp.sum((sk >= thresh).astype(jnp.int32),
                              axis=1, keepdims=True)
                return jnp.where(cnt >= k_eff, cand, t)

            t = jax.lax.fori_loop(0, 32, body,
                                  jnp.zeros((rows, 1), jnp.int32))
            thresh = t ^ sign
            m = sk >= thresh
            if width < s:
                m = jnp.concatenate(
                    [m, jnp.zeros((rows, s - width), jnp.bool_)], axis=1)
            mask_ref[0] = m


@functools.partial(jax.jit, static_argnames=())
def kernel(x, index_scores, Wq, Wk):
    bsz, s, _ = index_scores.shape
    k_fixed = min(_BASE_K, s)
    k_fixed = int(np.clip(k_fixed, _MIN_K, min(_MAX_K, s)))

    rows = min(_CHUNK, s)
    nchunks = s // rows
    grid = (bsz, nchunks)
    in_specs = [
        pl.BlockSpec((1, rows, (i + 1) * rows),
                     functools.partial(lambda i, b, r: (b, i, 0), i))
        for i in range(nchunks)
    ]
    mask = pl.pallas_call(
        functools.partial(_mask_kernel, k_fixed, nchunks),
        grid=grid,
        in_specs=in_specs,
        out_specs=pl.BlockSpec((1, rows, s), lambda b, r: (b, r, 0)),
        out_shape=jax.ShapeDtypeStruct((bsz, s, s), jnp.bool_),
    )(*([index_scores] * nchunks))

    k_values = jnp.full((bsz, s), k_fixed, dtype=jnp.int32)
    return (mask, k_values)
